# baseline (speedup 1.0000x reference)
import functools

import jax
import jax.numpy as jnp
from jax import lax
from jax.experimental import pallas as pl
from jax.experimental.pallas import tpu as pltpu
from jax.experimental.pallas import tpu_sc as plsc

_N = 10000
_E = 320000
_NC = 2
_NS = 16
_NW = _NC * _NS
_EPW = _E // _NW
_B = 80
_U = 32
_NCHUNK = _EPW // _B
_L = 16
_ROWS_PER_SUB = 624
_TAIL_START = _NS * _ROWS_PER_SUB
_TAIL_ROWS = _N - _TAIL_START


def _sc_edge_pass(xl, xr, src, dst, att, c_dim):
    C = c_dim
    mesh = plsc.VectorSubcoreMesh(core_axis_name="c", subcore_axis_name="s")

    @functools.partial(
        pl.kernel,
        out_type=[
            jax.ShapeDtypeStruct((_NC, _N, C), jnp.float32),
            jax.ShapeDtypeStruct((_NW, _N), jnp.float32),
        ],
        mesh=mesh,
        compiler_params=pltpu.CompilerParams(
            needs_layout_passes=False,
            use_tc_tiling_on_sc=(C % 128 == 0),
        ),
        scratch_types=[
            pltpu.VMEM((_B,), jnp.int32),
            pltpu.VMEM((_B,), jnp.int32),
            pltpu.VMEM((_B, C), jnp.float32),
            pltpu.VMEM((_B, C), jnp.float32),
            pltpu.VMEM((_B, C), jnp.float32),
            pltpu.VMEM((_N,), jnp.float32),
            pltpu.VMEM((C,), jnp.float32),
            pltpu.VMEM_SHARED((_N, C), jnp.float32),
            pltpu.SemaphoreType.DMA,
            pltpu.SemaphoreType.DMA,
        ],
    )
    def edge_kernel(xl_hbm, xr_hbm, src_hbm, dst_hbm, att_hbm, zrows_hbm,
                    zvec_hbm, acc_hbm, den_hbm,
                    idx_s, idx_d, rows_l, rows_r, rows_o, denom_tab, att_v,
                    acc_sh, sem1, sem2):
        cid = lax.axis_index("c")
        sid = lax.axis_index("s")
        wid = cid * _NS + sid

        pltpu.sync_copy(zvec_hbm, denom_tab)
        pltpu.sync_copy(zrows_hbm, acc_sh.at[pl.ds(sid * _ROWS_PER_SUB,
                                                   _ROWS_PER_SUB)])

        @pl.when(sid == _NS - 1)
        def _zero_tail():
            pltpu.sync_copy(zrows_hbm.at[pl.ds(0, _TAIL_ROWS)],
                            acc_sh.at[pl.ds(_TAIL_START, _TAIL_ROWS)])

        pltpu.sync_copy(att_hbm, att_v)
        plsc.subcore_barrier()

        def chunk_body(ch, carry):
            base = wid * _EPW + ch * _B
            pltpu.sync_copy(src_hbm.at[pl.ds(base, _B)], idx_s)
            pltpu.sync_copy(dst_hbm.at[pl.ds(base, _B)], idx_d)
            cp1 = pltpu.async_copy(xl_hbm.at[idx_s], rows_l, sem1)
            cp2 = pltpu.async_copy(xr_hbm.at[idx_d], rows_r, sem2)
            cp1.wait()
            cp2.wait()

            zv = jnp.zeros((_L,), jnp.float32)
            for g in range(_B // _L):
                eidx = g * _L + lax.iota(jnp.int32, _L)

                def logit_blk(b, accs):
                    base = jnp.full((_L,), b * _U, jnp.int32)
                    accs = list(accs)
                    for u in range(_U):
                        colv = base + u
                        vl = plsc.load_gather(rows_l, [eidx, colv])
                        vr = plsc.load_gather(rows_r, [eidx, colv])
                        a = plsc.load_gather(att_v, [colv])
                        s = vl + vr
                        accs[u % 4] = accs[u % 4] + a * jnp.maximum(s, 0.2 * s)
                    return tuple(accs)

                a0, a1, a2, a3 = lax.fori_loop(0, C // _U, logit_blk,
                                               (zv, zv, zv, zv))
                logit = (a0 + a1) + (a2 + a3)
                p = jnp.exp(logit)
                dstv = idx_d[pl.ds(g * _L, _L)]
                plsc.addupdate_scatter(denom_tab, [dstv], p)

                def scale_blk(b, carry2):
                    base = jnp.full((_L,), b * _U, jnp.int32)
                    for u in range(_U):
                        colv = base + u
                        v = plsc.load_gather(rows_l, [eidx, colv])
                        plsc.store_scatter(rows_o, [eidx, colv], v * p)
                    return carry2

                lax.fori_loop(0, C // _U, scale_blk, 0)

            pltpu.sync_copy(rows_o, acc_sh.at[idx_d], add=True)
            return carry

        lax.fori_loop(0, _NCHUNK, chunk_body, 0)

        pltpu.sync_copy(denom_tab, den_hbm.at[wid])
        plsc.subcore_barrier()
        r0 = sid * _ROWS_PER_SUB
        pltpu.sync_copy(acc_sh.at[pl.ds(r0, _ROWS_PER_SUB)],
                        acc_hbm.at[cid, pl.ds(r0, _ROWS_PER_SUB)])

        @pl.when(sid == _NS - 1)
        def _copy_tail():
            pltpu.sync_copy(acc_sh.at[pl.ds(_TAIL_START, _TAIL_ROWS)],
                            acc_hbm.at[cid, pl.ds(_TAIL_START, _TAIL_ROWS)])

    zrows = jnp.zeros((_ROWS_PER_SUB, C), jnp.float32)
    zvec = jnp.zeros((_N,), jnp.float32)
    return edge_kernel(xl, xr, src, dst, att, zrows, zvec)


def _tc_in_transform(x, wcat, bcat):
    nblk = 10
    bn = _N // nblk

    def body(x_ref, w_ref, b_ref, xl_ref, xr_ref, lin_ref):
        z = jnp.dot(x_ref[...], w_ref[...],
                    preferred_element_type=jnp.float32) + b_ref[...]
        xl_ref[...] = z[:, :128]
        xr_ref[...] = z[:, 128:256]
        lin_ref[...] = z[:, 256:]

    o = jax.ShapeDtypeStruct((_N, 128), jnp.float32)
    return pl.pallas_call(
        body,
        grid=(nblk,),
        in_specs=[
            pl.BlockSpec((bn, 128), lambda i: (i, 0)),
            pl.BlockSpec((128, 384), lambda i: (0, 0)),
            pl.BlockSpec((1, 384), lambda i: (0, 0)),
        ],
        out_specs=[pl.BlockSpec((bn, 128), lambda i: (i, 0))] * 3,
        out_shape=[o, o, o],
    )(x, wcat, bcat)


def _tc_mid(acc, dparts, lin, bias, wcat, bcat, c_in, c_out):
    nblk = 10
    bn = _N // nblk

    def body(a_ref, d_ref, l_ref, b_ref, w_ref, bc_ref,
             xl_ref, xr_ref, lin_ref):
        den = jnp.sum(d_ref[0], axis=0)
        g = (a_ref[0] + a_ref[1]) / (den[:, None] + 1e-16) + b_ref[...]
        h = jnp.maximum(g + l_ref[...], 0.0)
        z = jnp.dot(h, w_ref[...],
                    preferred_element_type=jnp.float32) + bc_ref[...]
        xl_ref[...] = z[:, :c_out]
        xr_ref[...] = z[:, c_out:2 * c_out]
        lin_ref[...] = z[:, 2 * c_out:]

    o = jax.ShapeDtypeStruct((_N, c_out), jnp.float32)
    return pl.pallas_call(
        body,
        grid=(nblk,),
        in_specs=[
            pl.BlockSpec((_NC, bn, c_in), lambda i: (0, i, 0)),
            pl.BlockSpec((1, _NW, bn), lambda i: (i, 0, 0)),
            pl.BlockSpec((bn, c_in), lambda i: (i, 0)),
            pl.BlockSpec((1, c_in), lambda i: (0, 0)),
            pl.BlockSpec((c_in, 3 * c_out), lambda i: (0, 0)),
            pl.BlockSpec((1, 3 * c_out), lambda i: (0, 0)),
        ],
        out_specs=[pl.BlockSpec((bn, c_out), lambda i: (i, 0))] * 3,
        out_shape=[o, o, o],
    )(acc, dparts, lin, bias, wcat, bcat)


def _tc_final(acc, dparts, lin, bias, w3, b3, c_in):
    nblk = 10
    bn = _N // nblk

    def body(a_ref, d_ref, l_ref, b_ref, w_ref, bc_ref, out_ref):
        den = jnp.sum(d_ref[0], axis=0)
        g = (a_ref[0] + a_ref[1]) / (den[:, None] + 1e-16) + b_ref[...]
        h = jnp.maximum(g + l_ref[...], 0.0)
        out_ref[...] = jnp.dot(h, w_ref[...],
                               preferred_element_type=jnp.float32) + bc_ref[...]

    return pl.pallas_call(
        body,
        grid=(nblk,),
        in_specs=[
            pl.BlockSpec((_NC, bn, c_in), lambda i: (0, i, 0)),
            pl.BlockSpec((1, _NW, bn), lambda i: (i, 0, 0)),
            pl.BlockSpec((bn, c_in), lambda i: (i, 0)),
            pl.BlockSpec((1, c_in), lambda i: (0, 0)),
            pl.BlockSpec((c_in, 1), lambda i: (0, 0)),
            pl.BlockSpec((1, 1), lambda i: (0, 0)),
        ],
        out_specs=pl.BlockSpec((bn, 1), lambda i: (i, 0)),
        out_shape=jax.ShapeDtypeStruct((_N, 1), jnp.float32),
    )(acc, dparts, lin, bias, w3, b3)


def kernel(x, edge_index, Wl1, bl1, Wr1, br1, att1, bias1, W1, b1,
           Wl2, bl2, Wr2, br2, att2, bias2, W2, b2, W3, b3):
    src = edge_index[0]
    dst = edge_index[1]

    wcat1 = jnp.concatenate([Wl1, Wr1, W1], axis=1)
    bcat1 = jnp.concatenate([bl1, br1, b1])[None, :]
    xl1, xr1, lin1 = _tc_in_transform(x, wcat1, bcat1)

    acc1, dp1 = _sc_edge_pass(xl1, xr1, src, dst, att1, 128)
    dp1_t = dp1.reshape(_NW, 10, _N // 10).swapaxes(0, 1)

    wcat2 = jnp.concatenate([Wl2, Wr2, W2], axis=1)
    bcat2 = jnp.concatenate([bl2, br2, b2])[None, :]
    xl2, xr2, lin2 = _tc_mid(acc1, dp1_t, lin1, bias1[None, :], wcat2, bcat2,
                             128, 64)

    acc2, dp2 = _sc_edge_pass(xl2, xr2, src, dst, att2, 64)
    dp2_t = dp2.reshape(_NW, 10, _N // 10).swapaxes(0, 1)

    return _tc_final(acc2, dp2_t, lin2, bias2[None, :], W3, b3[None, :], 64)


# speedup vs baseline: 5.3275x; 5.3275x over previous
import functools

import jax
import jax.numpy as jnp
from jax import lax
from jax.experimental import pallas as pl
from jax.experimental.pallas import tpu as pltpu
from jax.experimental.pallas import tpu_sc as plsc

_N = 10000
_E = 320000
_NC = 2
_NS = 16
_NW = _NC * _NS
_EPW = _E // _NW
_B = 80
_U = 32
_NCHUNK = _EPW // _B
_L = 16
_ROWS_PER_SUB = 624
_TAIL_START = _NS * _ROWS_PER_SUB
_TAIL_ROWS = _N - _TAIL_START


def _sc_edge_pass(xl, xr, src, dst, att, c_dim):
    C = c_dim
    mesh = plsc.VectorSubcoreMesh(core_axis_name="c", subcore_axis_name="s")

    @functools.partial(
        pl.kernel,
        out_type=[
            jax.ShapeDtypeStruct((_NC, _N, C), jnp.float32),
            jax.ShapeDtypeStruct((_NW, _N), jnp.float32),
        ],
        mesh=mesh,
        compiler_params=pltpu.CompilerParams(
            needs_layout_passes=False,
            use_tc_tiling_on_sc=(C % 128 == 0),
        ),
        scratch_types=[
            pltpu.VMEM((_B,), jnp.int32),
            pltpu.VMEM((_B,), jnp.int32),
            pltpu.VMEM((_B, C), jnp.float32),
            pltpu.VMEM((_B, C), jnp.float32),
            pltpu.VMEM((_B, C), jnp.float32),
            pltpu.VMEM((_N,), jnp.float32),
            pltpu.VMEM((C,), jnp.float32),
            pltpu.VMEM_SHARED((_N, C), jnp.float32),
            pltpu.SemaphoreType.DMA,
            pltpu.SemaphoreType.DMA,
        ],
    )
    def edge_kernel(xl_hbm, xr_hbm, src_hbm, dst_hbm, att_hbm, zrows_hbm,
                    zvec_hbm, acc_hbm, den_hbm,
                    idx_s, idx_d, rows_l, rows_r, rows_o, denom_tab, att_v,
                    acc_sh, sem1, sem2):
        cid = lax.axis_index("c")
        sid = lax.axis_index("s")
        wid = cid * _NS + sid

        pltpu.sync_copy(zvec_hbm, denom_tab)
        pltpu.sync_copy(zrows_hbm, acc_sh.at[pl.ds(sid * _ROWS_PER_SUB,
                                                   _ROWS_PER_SUB)])

        @pl.when(sid == _NS - 1)
        def _zero_tail():
            pltpu.sync_copy(zrows_hbm.at[pl.ds(0, _TAIL_ROWS)],
                            acc_sh.at[pl.ds(_TAIL_START, _TAIL_ROWS)])

        pltpu.sync_copy(att_hbm, att_v)
        plsc.subcore_barrier()

        lane = lax.iota(jnp.int32, _L)
        zv = jnp.zeros((_L,), jnp.float32)
        nk = C // _L
        attk = [att_v[pl.ds(k * _L, _L)] for k in range(nk)]

        def chunk_body(ch, carry):
            base = wid * _EPW + ch * _B
            pltpu.sync_copy(src_hbm.at[pl.ds(base, _B)], idx_s)
            pltpu.sync_copy(dst_hbm.at[pl.ds(base, _B)], idx_d)
            cp1 = pltpu.async_copy(xl_hbm.at[idx_s], rows_l, sem1)
            cp2 = pltpu.async_copy(xr_hbm.at[idx_d], rows_r, sem2)
            cp1.wait()
            cp2.wait()

            def grp_body(g, carry2):
                logits = zv
                for j in range(_L):
                    e = g * _L + j
                    accs = [zv, zv, zv, zv]
                    vls = []
                    for k in range(nk):
                        vl = rows_l[e, pl.ds(k * _L, _L)]
                        vr = rows_r[e, pl.ds(k * _L, _L)]
                        vls.append(vl)
                        s = vl + vr
                        accs[k % 4] = (accs[k % 4]
                                       + attk[k] * jnp.maximum(s, 0.2 * s))
                    lg = jnp.sum((accs[0] + accs[1]) + (accs[2] + accs[3]))
                    logits = jnp.where(lane == j, jnp.full((_L,), lg), logits)
                    pj = jnp.exp(jnp.full((_L,), lg))
                    for k in range(nk):
                        rows_o[e, pl.ds(k * _L, _L)] = vls[k] * pj
                p = jnp.exp(logits)
                dstv = idx_d[pl.ds(g * _L, _L)]
                plsc.addupdate_scatter(denom_tab, [dstv], p)
                return carry2

            lax.fori_loop(0, _B // _L, grp_body, 0)

            pltpu.sync_copy(rows_o, acc_sh.at[idx_d], add=True)
            return carry

        lax.fori_loop(0, _NCHUNK, chunk_body, 0)

        pltpu.sync_copy(denom_tab, den_hbm.at[wid])
        plsc.subcore_barrier()
        r0 = sid * _ROWS_PER_SUB
        pltpu.sync_copy(acc_sh.at[pl.ds(r0, _ROWS_PER_SUB)],
                        acc_hbm.at[cid, pl.ds(r0, _ROWS_PER_SUB)])

        @pl.when(sid == _NS - 1)
        def _copy_tail():
            pltpu.sync_copy(acc_sh.at[pl.ds(_TAIL_START, _TAIL_ROWS)],
                            acc_hbm.at[cid, pl.ds(_TAIL_START, _TAIL_ROWS)])

    zrows = jnp.zeros((_ROWS_PER_SUB, C), jnp.float32)
    zvec = jnp.zeros((_N,), jnp.float32)
    return edge_kernel(xl, xr, src, dst, att, zrows, zvec)


def _tc_in_transform(x, wcat, bcat):
    nblk = 10
    bn = _N // nblk

    def body(x_ref, w_ref, b_ref, xl_ref, xr_ref, lin_ref):
        z = jnp.dot(x_ref[...], w_ref[...],
                    preferred_element_type=jnp.float32) + b_ref[...]
        xl_ref[...] = z[:, :128]
        xr_ref[...] = z[:, 128:256]
        lin_ref[...] = z[:, 256:]

    o = jax.ShapeDtypeStruct((_N, 128), jnp.float32)
    return pl.pallas_call(
        body,
        grid=(nblk,),
        in_specs=[
            pl.BlockSpec((bn, 128), lambda i: (i, 0)),
            pl.BlockSpec((128, 384), lambda i: (0, 0)),
            pl.BlockSpec((1, 384), lambda i: (0, 0)),
        ],
        out_specs=[pl.BlockSpec((bn, 128), lambda i: (i, 0))] * 3,
        out_shape=[o, o, o],
    )(x, wcat, bcat)


def _tc_mid(acc, dparts, lin, bias, wcat, bcat, c_in, c_out):
    nblk = 10
    bn = _N // nblk

    def body(a_ref, d_ref, l_ref, b_ref, w_ref, bc_ref,
             xl_ref, xr_ref, lin_ref):
        den = jnp.sum(d_ref[0], axis=0)
        g = (a_ref[0] + a_ref[1]) / (den[:, None] + 1e-16) + b_ref[...]
        h = jnp.maximum(g + l_ref[...], 0.0)
        z = jnp.dot(h, w_ref[...],
                    preferred_element_type=jnp.float32) + bc_ref[...]
        xl_ref[...] = z[:, :c_out]
        xr_ref[...] = z[:, c_out:2 * c_out]
        lin_ref[...] = z[:, 2 * c_out:]

    o = jax.ShapeDtypeStruct((_N, c_out), jnp.float32)
    return pl.pallas_call(
        body,
        grid=(nblk,),
        in_specs=[
            pl.BlockSpec((_NC, bn, c_in), lambda i: (0, i, 0)),
            pl.BlockSpec((1, _NW, bn), lambda i: (i, 0, 0)),
            pl.BlockSpec((bn, c_in), lambda i: (i, 0)),
            pl.BlockSpec((1, c_in), lambda i: (0, 0)),
            pl.BlockSpec((c_in, 3 * c_out), lambda i: (0, 0)),
            pl.BlockSpec((1, 3 * c_out), lambda i: (0, 0)),
        ],
        out_specs=[pl.BlockSpec((bn, c_out), lambda i: (i, 0))] * 3,
        out_shape=[o, o, o],
    )(acc, dparts, lin, bias, wcat, bcat)


def _tc_final(acc, dparts, lin, bias, w3, b3, c_in):
    nblk = 10
    bn = _N // nblk

    def body(a_ref, d_ref, l_ref, b_ref, w_ref, bc_ref, out_ref):
        den = jnp.sum(d_ref[0], axis=0)
        g = (a_ref[0] + a_ref[1]) / (den[:, None] + 1e-16) + b_ref[...]
        h = jnp.maximum(g + l_ref[...], 0.0)
        out_ref[...] = jnp.dot(h, w_ref[...],
                               preferred_element_type=jnp.float32) + bc_ref[...]

    return pl.pallas_call(
        body,
        grid=(nblk,),
        in_specs=[
            pl.BlockSpec((_NC, bn, c_in), lambda i: (0, i, 0)),
            pl.BlockSpec((1, _NW, bn), lambda i: (i, 0, 0)),
            pl.BlockSpec((bn, c_in), lambda i: (i, 0)),
            pl.BlockSpec((1, c_in), lambda i: (0, 0)),
            pl.BlockSpec((c_in, 1), lambda i: (0, 0)),
            pl.BlockSpec((1, 1), lambda i: (0, 0)),
        ],
        out_specs=pl.BlockSpec((bn, 1), lambda i: (i, 0)),
        out_shape=jax.ShapeDtypeStruct((_N, 1), jnp.float32),
    )(acc, dparts, lin, bias, w3, b3)


def kernel(x, edge_index, Wl1, bl1, Wr1, br1, att1, bias1, W1, b1,
           Wl2, bl2, Wr2, br2, att2, bias2, W2, b2, W3, b3):
    src = edge_index[0]
    dst = edge_index[1]

    wcat1 = jnp.concatenate([Wl1, Wr1, W1], axis=1)
    bcat1 = jnp.concatenate([bl1, br1, b1])[None, :]
    xl1, xr1, lin1 = _tc_in_transform(x, wcat1, bcat1)

    acc1, dp1 = _sc_edge_pass(xl1, xr1, src, dst, att1, 128)
    dp1_t = dp1.reshape(_NW, 10, _N // 10).swapaxes(0, 1)

    wcat2 = jnp.concatenate([Wl2, Wr2, W2], axis=1)
    bcat2 = jnp.concatenate([bl2, br2, b2])[None, :]
    xl2, xr2, lin2 = _tc_mid(acc1, dp1_t, lin1, bias1[None, :], wcat2, bcat2,
                             128, 64)

    acc2, dp2 = _sc_edge_pass(xl2, xr2, src, dst, att2, 64)
    dp2_t = dp2.reshape(_NW, 10, _N // 10).swapaxes(0, 1)

    return _tc_final(acc2, dp2_t, lin2, bias2[None, :], W3, b3[None, :], 64)


# speedup vs baseline: 7.3457x; 1.3788x over previous
import functools

import jax
import jax.numpy as jnp
from jax import lax
from jax.experimental import pallas as pl
from jax.experimental.pallas import tpu as pltpu
from jax.experimental.pallas import tpu_sc as plsc

_N = 10000
_E = 320000
_NC = 2
_NS = 16
_NW = _NC * _NS
_EPW = _E // _NW
_B = 80
_U = 32
_NCHUNK = _EPW // _B
_L = 16
_ROWS_PER_SUB = 624
_TAIL_START = _NS * _ROWS_PER_SUB
_TAIL_ROWS = _N - _TAIL_START


def _sc_edge_pass(xl, xr, src, dst, att, c_dim):
    C = c_dim
    mesh = plsc.VectorSubcoreMesh(core_axis_name="c", subcore_axis_name="s")

    @functools.partial(
        pl.kernel,
        out_type=[
            jax.ShapeDtypeStruct((_NC, _N, C), jnp.float32),
            jax.ShapeDtypeStruct((_NW, _N), jnp.float32),
        ],
        mesh=mesh,
        compiler_params=pltpu.CompilerParams(
            needs_layout_passes=False,
            use_tc_tiling_on_sc=(C % 128 == 0),
        ),
        scratch_types=[
            pltpu.VMEM((_B,), jnp.int32),
            pltpu.VMEM((_B,), jnp.int32),
            pltpu.VMEM((_B,), jnp.int32),
            pltpu.VMEM((_B,), jnp.int32),
            pltpu.VMEM((_B, C), jnp.float32),
            pltpu.VMEM((_B, C), jnp.float32),
            pltpu.VMEM((_B, C), jnp.float32),
            pltpu.VMEM((_N,), jnp.float32),
            pltpu.VMEM((C,), jnp.float32),
            pltpu.VMEM_SHARED((_N, C), jnp.float32),
            pltpu.SemaphoreType.DMA,
            pltpu.SemaphoreType.DMA,
            pltpu.SemaphoreType.DMA,
            pltpu.SemaphoreType.DMA,
            pltpu.SemaphoreType.DMA,
        ],
    )
    def edge_kernel(xl_hbm, xr_hbm, src_hbm, dst_hbm, att_hbm, zrows_hbm,
                    zvec_hbm, acc_hbm, den_hbm,
                    idx_s0, idx_d0, idx_s1, idx_d1,
                    rows_l0, rows_l1, rows_r0, denom_tab,
                    att_v, acc_sh, semi0, semi1, seml0, seml1, semr0):
        cid = lax.axis_index("c")
        sid = lax.axis_index("s")
        wid = cid * _NS + sid

        pltpu.sync_copy(zvec_hbm, denom_tab)
        pltpu.sync_copy(zrows_hbm, acc_sh.at[pl.ds(sid * _ROWS_PER_SUB,
                                                   _ROWS_PER_SUB)])

        @pl.when(sid == _NS - 1)
        def _zero_tail():
            pltpu.sync_copy(zrows_hbm.at[pl.ds(0, _TAIL_ROWS)],
                            acc_sh.at[pl.ds(_TAIL_START, _TAIL_ROWS)])

        pltpu.sync_copy(att_hbm, att_v)
        plsc.subcore_barrier()

        lane = lax.iota(jnp.int32, _L)
        zv = jnp.zeros((_L,), jnp.float32)
        nk = C // _L
        attk = [att_v[pl.ds(k * _L, _L)] for k in range(nk)]

        idx_s = [idx_s0, idx_s1]
        idx_d = [idx_d0, idx_d1]
        rows_l = [rows_l0, rows_l1]
        semi = [semi0, semi1]
        seml = [seml0, seml1]

        def idx_start(ch, b):
            base = wid * _EPW + ch * _B
            pltpu.async_copy(src_hbm.at[pl.ds(base, _B)], idx_s[b], semi[b])
            pltpu.async_copy(dst_hbm.at[pl.ds(base, _B)], idx_d[b], semi[b])

        def idx_wait(ch, b):
            base = wid * _EPW + ch * _B
            pltpu.make_async_copy(src_hbm.at[pl.ds(base, _B)], idx_s[b],
                                  semi[b]).wait()
            pltpu.make_async_copy(dst_hbm.at[pl.ds(base, _B)], idx_d[b],
                                  semi[b]).wait()

        def gather_l_start(b):
            pltpu.async_copy(xl_hbm.at[idx_s[b]], rows_l[b], seml[b])

        def gather_l_wait(b):
            pltpu.make_async_copy(xl_hbm.at[idx_s[b]], rows_l[b],
                                  seml[b]).wait()

        def gather_r_start(b):
            pltpu.async_copy(xr_hbm.at[idx_d[b]], rows_r0, semr0)

        def gather_r_wait(b):
            pltpu.make_async_copy(xr_hbm.at[idx_d[b]], rows_r0,
                                  semr0).wait()

        def compute(b):
            rl, dv = rows_l[b], idx_d[b]

            def grp_body(g, carry2):
                logits = zv
                for j in range(_L):
                    e = g * _L + j
                    accs = [zv, zv, zv, zv]
                    vls = []
                    for k in range(nk):
                        vl = rl[e, pl.ds(k * _L, _L)]
                        vr = rows_r0[e, pl.ds(k * _L, _L)]
                        vls.append(vl)
                        s = vl + vr
                        accs[k % 4] = (accs[k % 4]
                                       + attk[k] * jnp.maximum(s, 0.2 * s))
                    lg = jnp.sum((accs[0] + accs[1]) + (accs[2] + accs[3]))
                    logits = jnp.where(lane == j, jnp.full((_L,), lg), logits)
                    pj = jnp.exp(jnp.full((_L,), lg))
                    for k in range(nk):
                        rows_r0[e, pl.ds(k * _L, _L)] = vls[k] * pj
                p = jnp.exp(logits)
                dstv = dv[pl.ds(g * _L, _L)]
                plsc.addupdate_scatter(denom_tab, [dstv], p)
                return carry2

            lax.fori_loop(0, _B // _L, grp_body, 0)

            pltpu.sync_copy(rows_r0, acc_sh.at[dv], add=True)

        idx_start(0, 0)
        idx_wait(0, 0)
        gather_l_start(0)
        gather_r_start(0)
        idx_start(1, 1)

        def pair_body(i, carry):
            ch = 2 * i
            idx_wait(ch + 1, 1)
            gather_l_start(1)
            gather_l_wait(0)
            gather_r_wait(0)
            compute(0)
            gather_r_start(1)
            idx_start(ch + 2, 0)
            idx_wait(ch + 2, 0)
            gather_l_start(0)
            gather_l_wait(1)
            gather_r_wait(1)
            compute(1)
            gather_r_start(0)

            @pl.when(i < _NCHUNK // 2 - 1)
            def _prefetch_idx():
                idx_start(ch + 3, 1)

            return carry

        lax.fori_loop(0, _NCHUNK // 2, pair_body, 0)
        gather_l_wait(0)
        gather_r_wait(0)
        compute(0)

        pltpu.sync_copy(denom_tab, den_hbm.at[wid])
        plsc.subcore_barrier()
        r0 = sid * _ROWS_PER_SUB
        pltpu.sync_copy(acc_sh.at[pl.ds(r0, _ROWS_PER_SUB)],
                        acc_hbm.at[cid, pl.ds(r0, _ROWS_PER_SUB)])

        @pl.when(sid == _NS - 1)
        def _copy_tail():
            pltpu.sync_copy(acc_sh.at[pl.ds(_TAIL_START, _TAIL_ROWS)],
                            acc_hbm.at[cid, pl.ds(_TAIL_START, _TAIL_ROWS)])

    zrows = jnp.zeros((_ROWS_PER_SUB, C), jnp.float32)
    zvec = jnp.zeros((_N,), jnp.float32)
    return edge_kernel(xl, xr, src, dst, att, zrows, zvec)


def _tc_in_transform(x, wcat, bcat):
    nblk = 10
    bn = _N // nblk

    def body(x_ref, w_ref, b_ref, xl_ref, xr_ref, lin_ref):
        z = jnp.dot(x_ref[...], w_ref[...],
                    preferred_element_type=jnp.float32) + b_ref[...]
        xl_ref[...] = z[:, :128]
        xr_ref[...] = z[:, 128:256]
        lin_ref[...] = z[:, 256:]

    o = jax.ShapeDtypeStruct((_N, 128), jnp.float32)
    return pl.pallas_call(
        body,
        grid=(nblk,),
        in_specs=[
            pl.BlockSpec((bn, 128), lambda i: (i, 0)),
            pl.BlockSpec((128, 384), lambda i: (0, 0)),
            pl.BlockSpec((1, 384), lambda i: (0, 0)),
        ],
        out_specs=[pl.BlockSpec((bn, 128), lambda i: (i, 0))] * 3,
        out_shape=[o, o, o],
    )(x, wcat, bcat)


def _tc_mid(acc, dparts, lin, bias, wcat, bcat, c_in, c_out):
    nblk = 10
    bn = _N // nblk

    def body(a_ref, d_ref, l_ref, b_ref, w_ref, bc_ref,
             xl_ref, xr_ref, lin_ref):
        den = jnp.sum(d_ref[0], axis=0)
        g = (a_ref[0] + a_ref[1]) / (den[:, None] + 1e-16) + b_ref[...]
        h = jnp.maximum(g + l_ref[...], 0.0)
        z = jnp.dot(h, w_ref[...],
                    preferred_element_type=jnp.float32) + bc_ref[...]
        xl_ref[...] = z[:, :c_out]
        xr_ref[...] = z[:, c_out:2 * c_out]
        lin_ref[...] = z[:, 2 * c_out:]

    o = jax.ShapeDtypeStruct((_N, c_out), jnp.float32)
    return pl.pallas_call(
        body,
        grid=(nblk,),
        in_specs=[
            pl.BlockSpec((_NC, bn, c_in), lambda i: (0, i, 0)),
            pl.BlockSpec((1, _NW, bn), lambda i: (i, 0, 0)),
            pl.BlockSpec((bn, c_in), lambda i: (i, 0)),
            pl.BlockSpec((1, c_in), lambda i: (0, 0)),
            pl.BlockSpec((c_in, 3 * c_out), lambda i: (0, 0)),
            pl.BlockSpec((1, 3 * c_out), lambda i: (0, 0)),
        ],
        out_specs=[pl.BlockSpec((bn, c_out), lambda i: (i, 0))] * 3,
        out_shape=[o, o, o],
    )(acc, dparts, lin, bias, wcat, bcat)


def _tc_final(acc, dparts, lin, bias, w3, b3, c_in):
    nblk = 10
    bn = _N // nblk

    def body(a_ref, d_ref, l_ref, b_ref, w_ref, bc_ref, out_ref):
        den = jnp.sum(d_ref[0], axis=0)
        g = (a_ref[0] + a_ref[1]) / (den[:, None] + 1e-16) + b_ref[...]
        h = jnp.maximum(g + l_ref[...], 0.0)
        out_ref[...] = jnp.dot(h, w_ref[...],
                               preferred_element_type=jnp.float32) + bc_ref[...]

    return pl.pallas_call(
        body,
        grid=(nblk,),
        in_specs=[
            pl.BlockSpec((_NC, bn, c_in), lambda i: (0, i, 0)),
            pl.BlockSpec((1, _NW, bn), lambda i: (i, 0, 0)),
            pl.BlockSpec((bn, c_in), lambda i: (i, 0)),
            pl.BlockSpec((1, c_in), lambda i: (0, 0)),
            pl.BlockSpec((c_in, 1), lambda i: (0, 0)),
            pl.BlockSpec((1, 1), lambda i: (0, 0)),
        ],
        out_specs=pl.BlockSpec((bn, 1), lambda i: (i, 0)),
        out_shape=jax.ShapeDtypeStruct((_N, 1), jnp.float32),
    )(acc, dparts, lin, bias, w3, b3)


def kernel(x, edge_index, Wl1, bl1, Wr1, br1, att1, bias1, W1, b1,
           Wl2, bl2, Wr2, br2, att2, bias2, W2, b2, W3, b3):
    src = edge_index[0]
    dst = edge_index[1]

    wcat1 = jnp.concatenate([Wl1, Wr1, W1], axis=1)
    bcat1 = jnp.concatenate([bl1, br1, b1])[None, :]
    xl1, xr1, lin1 = _tc_in_transform(x, wcat1, bcat1)

    acc1, dp1 = _sc_edge_pass(xl1, xr1, src, dst, att1, 128)
    dp1_t = dp1.reshape(_NW, 10, _N // 10).swapaxes(0, 1)

    wcat2 = jnp.concatenate([Wl2, Wr2, W2], axis=1)
    bcat2 = jnp.concatenate([bl2, br2, b2])[None, :]
    xl2, xr2, lin2 = _tc_mid(acc1, dp1_t, lin1, bias1[None, :], wcat2, bcat2,
                             128, 64)

    acc2, dp2 = _sc_edge_pass(xl2, xr2, src, dst, att2, 64)
    dp2_t = dp2.reshape(_NW, 10, _N // 10).swapaxes(0, 1)

    return _tc_final(acc2, dp2_t, lin2, bias2[None, :], W3, b3[None, :], 64)
